# Initial kernel scaffold; baseline (speedup 1.0000x reference)
import jax
import jax.numpy as jnp
from jax.experimental import pallas as pl


def kernel(x, edge_index, Wl1, Wr1, att1, b1, Wl2, Wr2, att2, b2):
    raise NotImplementedError("write your pallas kernel here")


# speedup vs baseline: 8.0082x; 8.0082x over previous
import functools

import jax
import jax.numpy as jnp
from jax import lax
from jax.experimental import pallas as pl
from jax.experimental.pallas import tpu as pltpu
from jax.experimental.pallas import tpu_sc as plsc

N = 10000
D = 128
E = 320000
E_TOT = E + N
NC, NS, L = 2, 16, 16
NW = NC * NS
C = 64
T_PER = 10368
NCHUNK = T_PER // C
TOT = NW * T_PER
W = 144
ROWS_PER_TILE = N // NS
_COPY_PATTERN = tuple((i * C, C) for i in range(ROWS_PER_TILE // C)) + (
    ((ROWS_PER_TILE // C) * C, ROWS_PER_TILE % C),)


_MESH = plsc.VectorSubcoreMesh(core_axis_name="c", subcore_axis_name="s")


@functools.partial(
    pl.kernel,
    out_type=jax.ShapeDtypeStruct((NC, N, W), jnp.float32),
    mesh=_MESH,
    compiler_params=pltpu.CompilerParams(use_tc_tiling_on_sc=False,
                                         needs_layout_passes=False),
    scratch_types=[
        pltpu.VMEM((C,), jnp.int32),
        pltpu.VMEM((C,), jnp.int32),
        pltpu.VMEM((C, D), jnp.float32),
        pltpu.VMEM((C, D), jnp.float32),
        pltpu.VMEM((C, W), jnp.float32),
        pltpu.VMEM((D,), jnp.float32),
        pltpu.VMEM_SHARED((N, W), jnp.float32),
        pltpu.SemaphoreType.DMA,
        pltpu.SemaphoreType.DMA,
    ],
)
def _sc_edge(xl_hbm, xr_hbm, att_hbm, src_hbm, dst_hbm, out_hbm,
             idx_s, idx_d, rows_l, rows_r, staged, att_v, acc, sem1, sem2):
    cid = lax.axis_index("c")
    sid = lax.axis_index("s")
    wid = sid * NC + cid
    base_t = wid * T_PER

    pltpu.sync_copy(att_hbm, att_v)

    def _zrow(i, carry):
        for j in range(W // L):
            staged[i, pl.ds(j * L, L)] = jnp.zeros((L,), jnp.float32)
        return carry
    lax.fori_loop(0, C, _zrow, 0)
    r0 = sid * ROWS_PER_TILE
    for off, ln in _COPY_PATTERN:
        pltpu.sync_copy(staged.at[pl.ds(0, ln)], acc.at[pl.ds(r0 + off, ln)])
    plsc.subcore_barrier()

    def chunk_body(ci, att_t):
        ebase = base_t + ci * C
        pltpu.sync_copy(src_hbm.at[pl.ds(ebase, C)], idx_s)
        pltpu.sync_copy(dst_hbm.at[pl.ds(ebase, C)], idx_d)
        cp1 = pltpu.async_copy(xl_hbm.at[idx_s], rows_l, sem1)
        cp2 = pltpu.async_copy(xr_hbm.at[idx_d], rows_r, sem2)
        cp1.wait()
        cp2.wait()

        def edge_body(e, att_tt):
            accv = jnp.zeros((L,), jnp.float32)
            a_regs = []
            for j in range(D // L):
                a = rows_l[e, pl.ds(j * L, L)]
                b = rows_r[e, pl.ds(j * L, L)]
                v = a + b
                t = jnp.maximum(v, 0.2 * v)
                accv = accv + t * att_tt[j]
                a_regs.append(a)
            logit = jnp.sum(accv)
            scale = jnp.where(ebase + e < E_TOT, 1.0, 0.0)
            pv = jnp.exp(jnp.full((L,), logit, jnp.float32)) * scale
            for j in range(D // L):
                staged[e, pl.ds(j * L, L)] = a_regs[j] * pv
            staged[e, pl.ds(D, L)] = pv
            return att_tt
        att_t = lax.fori_loop(0, C, edge_body, att_t)
        pltpu.sync_copy(staged, acc.at[idx_d], add=True)
        return att_t

    att_t0 = tuple(att_v[pl.ds(j * L, L)] for j in range(D // L))
    lax.fori_loop(0, NCHUNK, chunk_body, att_t0)

    plsc.subcore_barrier()
    for off, ln in _COPY_PATTERN:
        pltpu.sync_copy(acc.at[pl.ds(r0 + off, ln)],
                        out_hbm.at[cid, pl.ds(r0 + off, ln)])


_BR = 1000


def _mm2_body(x_ref, wl_ref, wr_ref, xl_ref, xr_ref):
    xb = x_ref[...]
    xl_ref[...] = jnp.dot(xb, wl_ref[...], preferred_element_type=jnp.float32)
    xr_ref[...] = jnp.dot(xb, wr_ref[...], preferred_element_type=jnp.float32)


def _mm2(x, wl, wr):
    return pl.pallas_call(
        _mm2_body,
        grid=(N // _BR,),
        in_specs=[
            pl.BlockSpec((_BR, D), lambda i: (i, 0)),
            pl.BlockSpec((D, D), lambda i: (0, 0)),
            pl.BlockSpec((D, D), lambda i: (0, 0)),
        ],
        out_specs=[
            pl.BlockSpec((_BR, D), lambda i: (i, 0)),
            pl.BlockSpec((_BR, D), lambda i: (i, 0)),
        ],
        out_shape=[
            jax.ShapeDtypeStruct((N, D), jnp.float32),
            jax.ShapeDtypeStruct((N, D), jnp.float32),
        ],
    )(x, wl, wr)


def _combine(p0, p1):
    num = p0[:, :D] + p1[:, :D]
    den = p0[:, D:D + 1] + p1[:, D:D + 1]
    return num / (den + 1e-16)


def _mid_body(p0_ref, p1_ref, b_ref, wl_ref, wr_ref, xl_ref, xr_ref):
    h = jnp.maximum(_combine(p0_ref[...], p1_ref[...]) + b_ref[...], 0.0)
    xl_ref[...] = jnp.dot(h, wl_ref[...], preferred_element_type=jnp.float32)
    xr_ref[...] = jnp.dot(h, wr_ref[...], preferred_element_type=jnp.float32)


def _mid(parts, b, wl, wr):
    return pl.pallas_call(
        _mid_body,
        grid=(N // _BR,),
        in_specs=[
            pl.BlockSpec((_BR, W), lambda i: (i, 0)),
            pl.BlockSpec((_BR, W), lambda i: (i, 0)),
            pl.BlockSpec((1, D), lambda i: (0, 0)),
            pl.BlockSpec((D, D), lambda i: (0, 0)),
            pl.BlockSpec((D, D), lambda i: (0, 0)),
        ],
        out_specs=[
            pl.BlockSpec((_BR, D), lambda i: (i, 0)),
            pl.BlockSpec((_BR, D), lambda i: (i, 0)),
        ],
        out_shape=[
            jax.ShapeDtypeStruct((N, D), jnp.float32),
            jax.ShapeDtypeStruct((N, D), jnp.float32),
        ],
    )(parts[0], parts[1], b.reshape(1, D), wl, wr)


def _fin_body(p0_ref, p1_ref, b_ref, o_ref):
    o_ref[...] = _combine(p0_ref[...], p1_ref[...]) + b_ref[...]


def _fin(parts, b):
    return pl.pallas_call(
        _fin_body,
        grid=(N // _BR,),
        in_specs=[
            pl.BlockSpec((_BR, W), lambda i: (i, 0)),
            pl.BlockSpec((_BR, W), lambda i: (i, 0)),
            pl.BlockSpec((1, D), lambda i: (0, 0)),
        ],
        out_specs=pl.BlockSpec((_BR, D), lambda i: (i, 0)),
        out_shape=jax.ShapeDtypeStruct((N, D), jnp.float32),
    )(parts[0], parts[1], b.reshape(1, D))


def kernel(x, edge_index, Wl1, Wr1, att1, b1, Wl2, Wr2, att2, b2):
    loop = jnp.arange(N, dtype=edge_index.dtype)
    pad = jnp.zeros((TOT - E_TOT,), dtype=edge_index.dtype)
    srcp = jnp.concatenate([edge_index[0], loop, pad])
    dstp = jnp.concatenate([edge_index[1], loop, pad])

    xl1, xr1 = _mm2(x, Wl1, Wr1)
    parts1 = _sc_edge(xl1, xr1, att1, srcp, dstp)
    xl2, xr2 = _mid(parts1, b1, Wl2, Wr2)
    parts2 = _sc_edge(xl2, xr2, att2, srcp, dstp)
    return _fin(parts2, b2)


# speedup vs baseline: 10.3648x; 1.2943x over previous
import functools

import jax
import jax.numpy as jnp
from jax import lax
from jax.experimental import pallas as pl
from jax.experimental.pallas import tpu as pltpu
from jax.experimental.pallas import tpu_sc as plsc

N = 10000
D = 128
E = 320000
E_TOT = E + N
NC, NS, L = 2, 16, 16
NW = NC * NS
C = 56
NCHUNK = 186
T_PER = NCHUNK * C
TOT = NW * T_PER
W = 144
ROWS_PER_TILE = N // NS
_COPY_PATTERN = tuple((i * C, C) for i in range(ROWS_PER_TILE // C)) + (
    ((ROWS_PER_TILE // C) * C, ROWS_PER_TILE % C),)


_MESH = plsc.VectorSubcoreMesh(core_axis_name="c", subcore_axis_name="s")


@functools.partial(
    pl.kernel,
    out_type=jax.ShapeDtypeStruct((NC, N, W), jnp.float32),
    mesh=_MESH,
    compiler_params=pltpu.CompilerParams(use_tc_tiling_on_sc=False,
                                         needs_layout_passes=False),
    scratch_types=[
        pltpu.VMEM((C,), jnp.int32),
        pltpu.VMEM((C,), jnp.int32),
        pltpu.VMEM((C,), jnp.int32),
        pltpu.VMEM((C,), jnp.int32),
        pltpu.VMEM((C, D), jnp.float32),
        pltpu.VMEM((C, D), jnp.float32),
        pltpu.VMEM((C, D), jnp.float32),
        pltpu.VMEM((C, D), jnp.float32),
        pltpu.VMEM((C, W), jnp.float32),
        pltpu.VMEM((D,), jnp.float32),
        pltpu.VMEM_SHARED((N, W), jnp.float32),
        pltpu.SemaphoreType.DMA,
        pltpu.SemaphoreType.DMA,
        pltpu.SemaphoreType.DMA,
        pltpu.SemaphoreType.DMA,
    ],
)
def _sc_edge(xl_hbm, xr_hbm, att_hbm, src_hbm, dst_hbm, out_hbm,
             is0, is1, id0, id1, rl0, rl1, rr0, rr1, staged, att_v, acc,
             sl0, sr0, sl1, sr1):
    cid = lax.axis_index("c")
    sid = lax.axis_index("s")
    wid = sid * NC + cid
    idx_s = (is0, is1)
    idx_d = (id0, id1)
    rows_l = (rl0, rl1)
    rows_r = (rr0, rr1)
    sem_l = (sl0, sl1)
    sem_r = (sr0, sr1)
    base_t = wid * T_PER

    pltpu.sync_copy(att_hbm, att_v)

    def _zrow(i, carry):
        for j in range(W // L):
            staged[i, pl.ds(j * L, L)] = jnp.zeros((L,), jnp.float32)
        return carry
    lax.fori_loop(0, C, _zrow, 0)
    r0 = sid * ROWS_PER_TILE
    for off, ln in _COPY_PATTERN:
        pltpu.sync_copy(staged.at[pl.ds(0, ln)], acc.at[pl.ds(r0 + off, ln)])
    plsc.subcore_barrier()

    def _load_issue(g, b):
        ebase = base_t + g * C
        pltpu.sync_copy(src_hbm.at[pl.ds(ebase, C)], idx_s[b])
        pltpu.sync_copy(dst_hbm.at[pl.ds(ebase, C)], idx_d[b])
        pltpu.async_copy(xl_hbm.at[idx_s[b]], rows_l[b], sem_l[b])
        pltpu.async_copy(xr_hbm.at[idx_d[b]], rows_r[b], sem_r[b])

    def _wait(b):
        pltpu.make_async_copy(xl_hbm.at[idx_s[b]], rows_l[b], sem_l[b]).wait()
        pltpu.make_async_copy(xr_hbm.at[idx_d[b]], rows_r[b], sem_r[b]).wait()

    _load_issue(0, 0)
    att_t0 = tuple(att_v[pl.ds(j * L, L)] for j in range(D // L))

    def pair_body(i, att_t):
        for b in range(2):
            g = 2 * i + b
            gnext = jnp.where(g + 1 == NCHUNK, 0, g + 1)
            _load_issue(gnext, (b + 1) % 2)
            _wait(b)
            ebase = base_t + g * C

            def edge_body(e, att_tt):
                accv = jnp.zeros((L,), jnp.float32)
                a_regs = []
                for j in range(D // L):
                    a = rows_l[b][e, pl.ds(j * L, L)]
                    bb = rows_r[b][e, pl.ds(j * L, L)]
                    v = a + bb
                    t = jnp.maximum(v, 0.2 * v)
                    accv = accv + t * att_tt[j]
                    a_regs.append(a)
                logit = jnp.sum(accv)
                scale = jnp.where(ebase + e < E_TOT, 1.0, 0.0)
                pv = jnp.exp(jnp.full((L,), logit, jnp.float32)) * scale
                for j in range(D // L):
                    staged[e, pl.ds(j * L, L)] = a_regs[j] * pv
                staged[e, pl.ds(D, L)] = pv
                return att_tt
            att_t = lax.fori_loop(0, C, edge_body, att_t)
            pltpu.sync_copy(staged, acc.at[idx_d[b]], add=True)
        return att_t

    lax.fori_loop(0, NCHUNK // 2, pair_body, att_t0)
    _wait(0)

    plsc.subcore_barrier()
    for off, ln in _COPY_PATTERN:
        pltpu.sync_copy(acc.at[pl.ds(r0 + off, ln)],
                        out_hbm.at[cid, pl.ds(r0 + off, ln)])


_BR = 1000


def _mm2_body(x_ref, wl_ref, wr_ref, xl_ref, xr_ref):
    xb = x_ref[...]
    xl_ref[...] = jnp.dot(xb, wl_ref[...], preferred_element_type=jnp.float32)
    xr_ref[...] = jnp.dot(xb, wr_ref[...], preferred_element_type=jnp.float32)


def _mm2(x, wl, wr):
    return pl.pallas_call(
        _mm2_body,
        grid=(N // _BR,),
        in_specs=[
            pl.BlockSpec((_BR, D), lambda i: (i, 0)),
            pl.BlockSpec((D, D), lambda i: (0, 0)),
            pl.BlockSpec((D, D), lambda i: (0, 0)),
        ],
        out_specs=[
            pl.BlockSpec((_BR, D), lambda i: (i, 0)),
            pl.BlockSpec((_BR, D), lambda i: (i, 0)),
        ],
        out_shape=[
            jax.ShapeDtypeStruct((N, D), jnp.float32),
            jax.ShapeDtypeStruct((N, D), jnp.float32),
        ],
    )(x, wl, wr)


def _combine(p0, p1):
    num = p0[:, :D] + p1[:, :D]
    den = p0[:, D:D + 1] + p1[:, D:D + 1]
    return num / (den + 1e-16)


def _mid_body(p0_ref, p1_ref, b_ref, wl_ref, wr_ref, xl_ref, xr_ref):
    h = jnp.maximum(_combine(p0_ref[...], p1_ref[...]) + b_ref[...], 0.0)
    xl_ref[...] = jnp.dot(h, wl_ref[...], preferred_element_type=jnp.float32)
    xr_ref[...] = jnp.dot(h, wr_ref[...], preferred_element_type=jnp.float32)


def _mid(parts, b, wl, wr):
    return pl.pallas_call(
        _mid_body,
        grid=(N // _BR,),
        in_specs=[
            pl.BlockSpec((_BR, W), lambda i: (i, 0)),
            pl.BlockSpec((_BR, W), lambda i: (i, 0)),
            pl.BlockSpec((1, D), lambda i: (0, 0)),
            pl.BlockSpec((D, D), lambda i: (0, 0)),
            pl.BlockSpec((D, D), lambda i: (0, 0)),
        ],
        out_specs=[
            pl.BlockSpec((_BR, D), lambda i: (i, 0)),
            pl.BlockSpec((_BR, D), lambda i: (i, 0)),
        ],
        out_shape=[
            jax.ShapeDtypeStruct((N, D), jnp.float32),
            jax.ShapeDtypeStruct((N, D), jnp.float32),
        ],
    )(parts[0], parts[1], b.reshape(1, D), wl, wr)


def _fin_body(p0_ref, p1_ref, b_ref, o_ref):
    o_ref[...] = _combine(p0_ref[...], p1_ref[...]) + b_ref[...]


def _fin(parts, b):
    return pl.pallas_call(
        _fin_body,
        grid=(N // _BR,),
        in_specs=[
            pl.BlockSpec((_BR, W), lambda i: (i, 0)),
            pl.BlockSpec((_BR, W), lambda i: (i, 0)),
            pl.BlockSpec((1, D), lambda i: (0, 0)),
        ],
        out_specs=pl.BlockSpec((_BR, D), lambda i: (i, 0)),
        out_shape=jax.ShapeDtypeStruct((N, D), jnp.float32),
    )(parts[0], parts[1], b.reshape(1, D))


def kernel(x, edge_index, Wl1, Wr1, att1, b1, Wl2, Wr2, att2, b2):
    loop = jnp.arange(N, dtype=edge_index.dtype)
    pad = jnp.zeros((TOT - E_TOT,), dtype=edge_index.dtype)
    srcp = jnp.concatenate([edge_index[0], loop, pad])
    dstp = jnp.concatenate([edge_index[1], loop, pad])

    xl1, xr1 = _mm2(x, Wl1, Wr1)
    parts1 = _sc_edge(xl1, xr1, att1, srcp, dstp)
    xl2, xr2 = _mid(parts1, b1, Wl2, Wr2)
    parts2 = _sc_edge(xl2, xr2, att2, srcp, dstp)
    return _fin(parts2, b2)


# speedup vs baseline: 10.6447x; 1.0270x over previous
import functools

import jax
import jax.numpy as jnp
from jax import lax
from jax.experimental import pallas as pl
from jax.experimental.pallas import tpu as pltpu
from jax.experimental.pallas import tpu_sc as plsc

N = 10000
D = 128
E = 320000
E_TOT = E + N
NC, NS, L = 2, 16, 16
NW = NC * NS
C = 48
NCHUNK = 216
T_PER = NCHUNK * C
TOT = NW * T_PER
W = 144
ROWS_PER_TILE = N // NS
_COPY_PATTERN = tuple((i * C, C) for i in range(ROWS_PER_TILE // C)) + (
    ((ROWS_PER_TILE // C) * C, ROWS_PER_TILE % C),)


_MESH = plsc.VectorSubcoreMesh(core_axis_name="c", subcore_axis_name="s")


@functools.partial(
    pl.kernel,
    out_type=jax.ShapeDtypeStruct((NC, N, W), jnp.float32),
    mesh=_MESH,
    compiler_params=pltpu.CompilerParams(use_tc_tiling_on_sc=False,
                                         needs_layout_passes=False),
    scratch_types=[
        pltpu.VMEM((2, C), jnp.int32),
        pltpu.VMEM((2, C), jnp.int32),
        pltpu.VMEM((2, C), jnp.int32),
        pltpu.VMEM((2, C), jnp.int32),
        pltpu.VMEM((C, D), jnp.float32),
        pltpu.VMEM((C, D), jnp.float32),
        pltpu.VMEM((C, D), jnp.float32),
        pltpu.VMEM((C, D), jnp.float32),
        pltpu.VMEM((C, W), jnp.float32),
        pltpu.VMEM((C, W), jnp.float32),
        pltpu.VMEM((D,), jnp.float32),
        pltpu.VMEM_SHARED((N, W), jnp.float32),
        pltpu.SemaphoreType.DMA,
        pltpu.SemaphoreType.DMA,
        pltpu.SemaphoreType.DMA,
        pltpu.SemaphoreType.DMA,
        pltpu.SemaphoreType.DMA,
        pltpu.SemaphoreType.DMA,
        pltpu.SemaphoreType.DMA,
        pltpu.SemaphoreType.DMA,
        pltpu.SemaphoreType.DMA,
        pltpu.SemaphoreType.DMA,
    ],
)
def _sc_edge(xl_hbm, xr_hbm, att_hbm, idx_hbm, out_hbm,
             ix0, ix1, ix2, ix3, rl0, rl1, rr0, rr1, st0, st1, att_v, acc,
             si0, si1, si2, si3, sl0, sl1, sr0, sr1, ss0, ss1):
    cid = lax.axis_index("c")
    sid = lax.axis_index("s")
    wid = sid * NC + cid
    idxb = (ix0, ix1, ix2, ix3)
    sem_i = (si0, si1, si2, si3)
    rows_l = (rl0, rl1)
    rows_r = (rr0, rr1)
    sem_l = (sl0, sl1)
    sem_r = (sr0, sr1)
    staged = (st0, st1)
    sem_s = (ss0, ss1)
    base_t = wid * T_PER
    base_g = wid * NCHUNK

    def _wrap(g):
        return jnp.where(g >= NCHUNK, g - NCHUNK, g)

    def _idx_issue(g, k):
        pltpu.async_copy(idx_hbm.at[base_g + _wrap(g)], idxb[k], sem_i[k])

    def _idx_wait(g, k):
        pltpu.make_async_copy(idx_hbm.at[base_g + _wrap(g)], idxb[k],
                              sem_i[k]).wait()

    def _gather_issue(k, b):
        pltpu.async_copy(xl_hbm.at[idxb[k].at[0]], rows_l[b], sem_l[b])
        pltpu.async_copy(xr_hbm.at[idxb[k].at[1]], rows_r[b], sem_r[b])

    def _gather_wait(k, b):
        pltpu.make_async_copy(xl_hbm.at[idxb[k].at[0]], rows_l[b],
                              sem_l[b]).wait()
        pltpu.make_async_copy(xr_hbm.at[idxb[k].at[1]], rows_r[b],
                              sem_r[b]).wait()

    def _scatter_issue(k, b):
        pltpu.async_copy(staged[b], acc.at[idxb[k].at[1]], sem_s[b], add=True)

    def _scatter_wait(k, b):
        pltpu.make_async_copy(staged[b], acc.at[idxb[k].at[1]],
                              sem_s[b]).wait()

    pltpu.sync_copy(att_hbm, att_v)
    _idx_issue(0, 0)
    _idx_wait(0, 0)
    _gather_issue(0, 0)
    _idx_issue(1, 1)

    def _zrow(i, carry):
        for j in range(W // L):
            st0[i, pl.ds(j * L, L)] = jnp.zeros((L,), jnp.float32)
            st1[i, pl.ds(j * L, L)] = jnp.zeros((L,), jnp.float32)
        return carry
    lax.fori_loop(0, C, _zrow, 0)
    r0 = sid * ROWS_PER_TILE
    for off, ln in _COPY_PATTERN:
        pltpu.sync_copy(st0.at[pl.ds(0, ln)], acc.at[pl.ds(r0 + off, ln)])
    plsc.subcore_barrier()
    _scatter_issue(0, 0)
    _scatter_issue(0, 1)

    att_t0 = tuple(att_v[pl.ds(j * L, L)] for j in range(D // L))

    def quad_body(i, att_t):
        for u in range(4):
            g = 4 * i + u
            b = u % 2
            kcur = u
            knext = (u + 1) % 4
            _idx_wait(g + 1, knext)
            _gather_issue(knext, (b + 1) % 2)
            _scatter_wait(kcur, b)
            _gather_wait(kcur, b)
            ebase = base_t + g * C

            def edge_body(e, att_tt):
                accv = jnp.zeros((L,), jnp.float32)
                a_regs = []
                for j in range(D // L):
                    a = rows_l[b][e, pl.ds(j * L, L)]
                    bb = rows_r[b][e, pl.ds(j * L, L)]
                    v = a + bb
                    t = jnp.maximum(v, 0.2 * v)
                    accv = accv + t * att_tt[j]
                    a_regs.append(a)
                logit = jnp.where(ebase + e < E_TOT, jnp.sum(accv), -1e30)
                pv = jnp.exp(jnp.full((L,), logit, jnp.float32))
                for j in range(D // L):
                    staged[b][e, pl.ds(j * L, L)] = a_regs[j] * pv
                staged[b][e, pl.ds(D, L)] = pv
                return att_tt
            att_t = lax.fori_loop(0, C, edge_body, att_t)
            _scatter_issue(kcur, b)
            _idx_issue(g + 2, (u + 2) % 4)
        return att_t

    lax.fori_loop(0, NCHUNK // 4, quad_body, att_t0)
    _idx_wait(1, 1)
    _gather_wait(0, 0)
    _scatter_wait(2, 0)
    _scatter_wait(3, 1)

    plsc.subcore_barrier()
    for off, ln in _COPY_PATTERN:
        pltpu.sync_copy(acc.at[pl.ds(r0 + off, ln)],
                        out_hbm.at[cid, pl.ds(r0 + off, ln)])


_BR = 1000


def _mm2_body(x_ref, wl_ref, wr_ref, xl_ref, xr_ref):
    xb = x_ref[...]
    xl_ref[...] = jnp.dot(xb, wl_ref[...], preferred_element_type=jnp.float32)
    xr_ref[...] = jnp.dot(xb, wr_ref[...], preferred_element_type=jnp.float32)


def _mm2(x, wl, wr):
    return pl.pallas_call(
        _mm2_body,
        grid=(N // _BR,),
        in_specs=[
            pl.BlockSpec((_BR, D), lambda i: (i, 0)),
            pl.BlockSpec((D, D), lambda i: (0, 0)),
            pl.BlockSpec((D, D), lambda i: (0, 0)),
        ],
        out_specs=[
            pl.BlockSpec((_BR, D), lambda i: (i, 0)),
            pl.BlockSpec((_BR, D), lambda i: (i, 0)),
        ],
        out_shape=[
            jax.ShapeDtypeStruct((N, D), jnp.float32),
            jax.ShapeDtypeStruct((N, D), jnp.float32),
        ],
    )(x, wl, wr)


def _combine(p0, p1):
    num = p0[:, :D] + p1[:, :D]
    den = p0[:, D:D + 1] + p1[:, D:D + 1]
    return num / (den + 1e-16)


def _mid_body(p0_ref, p1_ref, b_ref, wl_ref, wr_ref, xl_ref, xr_ref):
    h = jnp.maximum(_combine(p0_ref[...], p1_ref[...]) + b_ref[...], 0.0)
    xl_ref[...] = jnp.dot(h, wl_ref[...], preferred_element_type=jnp.float32)
    xr_ref[...] = jnp.dot(h, wr_ref[...], preferred_element_type=jnp.float32)


def _mid(parts, b, wl, wr):
    return pl.pallas_call(
        _mid_body,
        grid=(N // _BR,),
        in_specs=[
            pl.BlockSpec((_BR, W), lambda i: (i, 0)),
            pl.BlockSpec((_BR, W), lambda i: (i, 0)),
            pl.BlockSpec((1, D), lambda i: (0, 0)),
            pl.BlockSpec((D, D), lambda i: (0, 0)),
            pl.BlockSpec((D, D), lambda i: (0, 0)),
        ],
        out_specs=[
            pl.BlockSpec((_BR, D), lambda i: (i, 0)),
            pl.BlockSpec((_BR, D), lambda i: (i, 0)),
        ],
        out_shape=[
            jax.ShapeDtypeStruct((N, D), jnp.float32),
            jax.ShapeDtypeStruct((N, D), jnp.float32),
        ],
    )(parts[0], parts[1], b.reshape(1, D), wl, wr)


def _fin_body(p0_ref, p1_ref, b_ref, o_ref):
    o_ref[...] = _combine(p0_ref[...], p1_ref[...]) + b_ref[...]


def _fin(parts, b):
    return pl.pallas_call(
        _fin_body,
        grid=(N // _BR,),
        in_specs=[
            pl.BlockSpec((_BR, W), lambda i: (i, 0)),
            pl.BlockSpec((_BR, W), lambda i: (i, 0)),
            pl.BlockSpec((1, D), lambda i: (0, 0)),
        ],
        out_specs=pl.BlockSpec((_BR, D), lambda i: (i, 0)),
        out_shape=jax.ShapeDtypeStruct((N, D), jnp.float32),
    )(parts[0], parts[1], b.reshape(1, D))


def kernel(x, edge_index, Wl1, Wr1, att1, b1, Wl2, Wr2, att2, b2):
    loop = jnp.arange(N, dtype=edge_index.dtype)
    pad = jnp.zeros((TOT - E_TOT,), dtype=edge_index.dtype)
    srcp = jnp.concatenate([edge_index[0], loop, pad]).reshape(TOT // C, C)
    dstp = jnp.concatenate([edge_index[1], loop, pad]).reshape(TOT // C, C)
    idx_pack = jnp.stack([srcp, dstp], axis=1)

    xl1, xr1 = _mm2(x, Wl1, Wr1)
    parts1 = _sc_edge(xl1, xr1, att1, idx_pack)
    xl2, xr2 = _mid(parts1, b1, Wl2, Wr2)
    parts2 = _sc_edge(xl2, xr2, att2, idx_pack)
    return _fin(parts2, b2)


# speedup vs baseline: 13.2256x; 1.2425x over previous
import functools

import jax
import jax.numpy as jnp
from jax import lax
from jax.experimental import pallas as pl
from jax.experimental.pallas import tpu as pltpu
from jax.experimental.pallas import tpu_sc as plsc

N = 10000
D = 128
E = 320000
E_TOT = E + N
NC, NS, L = 2, 16, 16
NW = NC * NS
C = 48
NCHUNK = 216
T_PER = NCHUNK * C
TOT = NW * T_PER
W = 144
ROWS_PER_TILE = N // NS
_COPY_PATTERN = tuple((i * C, C) for i in range(ROWS_PER_TILE // C)) + (
    ((ROWS_PER_TILE // C) * C, ROWS_PER_TILE % C),)


_MESH = plsc.VectorSubcoreMesh(core_axis_name="c", subcore_axis_name="s")


@functools.partial(
    pl.kernel,
    out_type=jax.ShapeDtypeStruct((NC, N, W), jnp.float32),
    mesh=_MESH,
    compiler_params=pltpu.CompilerParams(use_tc_tiling_on_sc=False,
                                         needs_layout_passes=False),
    scratch_types=[
        pltpu.VMEM((2, C), jnp.int32),
        pltpu.VMEM((2, C), jnp.int32),
        pltpu.VMEM((2, C), jnp.int32),
        pltpu.VMEM((2, C), jnp.int32),
        pltpu.VMEM((C, D), jnp.float32),
        pltpu.VMEM((C, D), jnp.float32),
        pltpu.VMEM((C, D), jnp.float32),
        pltpu.VMEM((C, D), jnp.float32),
        pltpu.VMEM((C, W), jnp.float32),
        pltpu.VMEM((C, W), jnp.float32),
        pltpu.VMEM((D,), jnp.float32),
        pltpu.VMEM_SHARED((N, W), jnp.float32),
        pltpu.SemaphoreType.DMA,
        pltpu.SemaphoreType.DMA,
        pltpu.SemaphoreType.DMA,
        pltpu.SemaphoreType.DMA,
        pltpu.SemaphoreType.DMA,
        pltpu.SemaphoreType.DMA,
        pltpu.SemaphoreType.DMA,
        pltpu.SemaphoreType.DMA,
        pltpu.SemaphoreType.DMA,
        pltpu.SemaphoreType.DMA,
    ],
)
def _sc_edge(xl_hbm, xr_hbm, att_hbm, idx_hbm, out_hbm,
             ix0, ix1, ix2, ix3, rl0, rl1, rr0, rr1, st0, st1, att_v, acc,
             si0, si1, si2, si3, sl0, sl1, sr0, sr1, ss0, ss1):
    cid = lax.axis_index("c")
    sid = lax.axis_index("s")
    wid = sid * NC + cid
    idxb = (ix0, ix1, ix2, ix3)
    sem_i = (si0, si1, si2, si3)
    rows_l = (rl0, rl1)
    rows_r = (rr0, rr1)
    sem_l = (sl0, sl1)
    sem_r = (sr0, sr1)
    staged = (st0, st1)
    sem_s = (ss0, ss1)
    base_t = wid * T_PER
    base_g = wid * NCHUNK

    def _wrap(g):
        return jnp.where(g >= NCHUNK, g - NCHUNK, g)

    def _idx_issue(g, k):
        pltpu.async_copy(idx_hbm.at[base_g + _wrap(g)], idxb[k], sem_i[k])

    def _idx_wait(g, k):
        pltpu.make_async_copy(idx_hbm.at[base_g + _wrap(g)], idxb[k],
                              sem_i[k]).wait()

    def _gather_issue(k, b):
        pltpu.async_copy(xl_hbm.at[idxb[k].at[0]], rows_l[b], sem_l[b])
        pltpu.async_copy(xr_hbm.at[idxb[k].at[1]], rows_r[b], sem_r[b])

    def _gather_wait(k, b):
        pltpu.make_async_copy(xl_hbm.at[idxb[k].at[0]], rows_l[b],
                              sem_l[b]).wait()
        pltpu.make_async_copy(xr_hbm.at[idxb[k].at[1]], rows_r[b],
                              sem_r[b]).wait()

    def _scatter_issue(k, b):
        pltpu.async_copy(staged[b], acc.at[idxb[k].at[1]], sem_s[b], add=True)

    def _scatter_wait(k, b):
        pltpu.make_async_copy(staged[b], acc.at[idxb[k].at[1]],
                              sem_s[b]).wait()

    pltpu.sync_copy(att_hbm, att_v)
    _idx_issue(0, 0)
    _idx_wait(0, 0)
    _gather_issue(0, 0)
    _idx_issue(1, 1)

    def _zrow(i, carry):
        for j in range(W // L):
            st0[i, pl.ds(j * L, L)] = jnp.zeros((L,), jnp.float32)
            st1[i, pl.ds(j * L, L)] = jnp.zeros((L,), jnp.float32)
        return carry
    lax.fori_loop(0, C, _zrow, 0)
    r0 = sid * ROWS_PER_TILE
    for off, ln in _COPY_PATTERN:
        pltpu.sync_copy(st0.at[pl.ds(0, ln)], acc.at[pl.ds(r0 + off, ln)])
    plsc.subcore_barrier()
    _scatter_issue(0, 0)
    _scatter_issue(0, 1)

    att_t0 = tuple(att_v[pl.ds(j * L, L)] for j in range(D // L))

    def quad_body(i, att_t):
        for u in range(4):
            g = 4 * i + u
            b = u % 2
            kcur = u
            knext = (u + 1) % 4
            _idx_wait(g + 1, knext)
            _gather_issue(knext, (b + 1) % 2)
            _scatter_wait(kcur, b)
            _gather_wait(kcur, b)
            ebase = base_t + g * C

            def edge_body(eh, att_tt):
                regs = []
                for s in range(2):
                    e = 2 * eh + s
                    acc0 = jnp.zeros((L,), jnp.float32)
                    acc1 = jnp.zeros((L,), jnp.float32)
                    a_regs = []
                    for j in range(D // L):
                        a = rows_l[b][e, pl.ds(j * L, L)]
                        bb = rows_r[b][e, pl.ds(j * L, L)]
                        v = a + bb
                        t = jnp.maximum(v, 0.2 * v)
                        if j % 2 == 0:
                            acc0 = acc0 + t * att_tt[j]
                        else:
                            acc1 = acc1 + t * att_tt[j]
                        a_regs.append(a)
                    logit = jnp.where(ebase + e < E_TOT,
                                      jnp.sum(acc0 + acc1), -1e30)
                    pv = jnp.exp(jnp.full((L,), logit, jnp.float32))
                    regs.append((e, a_regs, pv))
                for e, a_regs, pv in regs:
                    for j in range(D // L):
                        staged[b][e, pl.ds(j * L, L)] = a_regs[j] * pv
                    staged[b][e, pl.ds(D, L)] = pv
                return att_tt
            att_t = lax.fori_loop(0, C // 2, edge_body, att_t)
            _scatter_issue(kcur, b)
            _idx_issue(g + 2, (u + 2) % 4)
        return att_t

    lax.fori_loop(0, NCHUNK // 4, quad_body, att_t0)
    _idx_wait(1, 1)
    _gather_wait(0, 0)
    _scatter_wait(2, 0)
    _scatter_wait(3, 1)

    plsc.subcore_barrier()
    for off, ln in _COPY_PATTERN:
        pltpu.sync_copy(acc.at[pl.ds(r0 + off, ln)],
                        out_hbm.at[cid, pl.ds(r0 + off, ln)])


_BR = 1000


def _mm2_body(x_ref, wl_ref, wr_ref, xl_ref, xr_ref):
    xb = x_ref[...]
    xl_ref[...] = jnp.dot(xb, wl_ref[...], preferred_element_type=jnp.float32)
    xr_ref[...] = jnp.dot(xb, wr_ref[...], preferred_element_type=jnp.float32)


def _mm2(x, wl, wr):
    return pl.pallas_call(
        _mm2_body,
        grid=(N // _BR,),
        in_specs=[
            pl.BlockSpec((_BR, D), lambda i: (i, 0)),
            pl.BlockSpec((D, D), lambda i: (0, 0)),
            pl.BlockSpec((D, D), lambda i: (0, 0)),
        ],
        out_specs=[
            pl.BlockSpec((_BR, D), lambda i: (i, 0)),
            pl.BlockSpec((_BR, D), lambda i: (i, 0)),
        ],
        out_shape=[
            jax.ShapeDtypeStruct((N, D), jnp.float32),
            jax.ShapeDtypeStruct((N, D), jnp.float32),
        ],
    )(x, wl, wr)


def _combine(p0, p1):
    num = p0[:, :D] + p1[:, :D]
    den = p0[:, D:D + 1] + p1[:, D:D + 1]
    return num / (den + 1e-16)


def _mid_body(p0_ref, p1_ref, b_ref, wl_ref, wr_ref, xl_ref, xr_ref):
    h = jnp.maximum(_combine(p0_ref[...], p1_ref[...]) + b_ref[...], 0.0)
    xl_ref[...] = jnp.dot(h, wl_ref[...], preferred_element_type=jnp.float32)
    xr_ref[...] = jnp.dot(h, wr_ref[...], preferred_element_type=jnp.float32)


def _mid(parts, b, wl, wr):
    return pl.pallas_call(
        _mid_body,
        grid=(N // _BR,),
        in_specs=[
            pl.BlockSpec((_BR, W), lambda i: (i, 0)),
            pl.BlockSpec((_BR, W), lambda i: (i, 0)),
            pl.BlockSpec((1, D), lambda i: (0, 0)),
            pl.BlockSpec((D, D), lambda i: (0, 0)),
            pl.BlockSpec((D, D), lambda i: (0, 0)),
        ],
        out_specs=[
            pl.BlockSpec((_BR, D), lambda i: (i, 0)),
            pl.BlockSpec((_BR, D), lambda i: (i, 0)),
        ],
        out_shape=[
            jax.ShapeDtypeStruct((N, D), jnp.float32),
            jax.ShapeDtypeStruct((N, D), jnp.float32),
        ],
    )(parts[0], parts[1], b.reshape(1, D), wl, wr)


def _fin_body(p0_ref, p1_ref, b_ref, o_ref):
    o_ref[...] = _combine(p0_ref[...], p1_ref[...]) + b_ref[...]


def _fin(parts, b):
    return pl.pallas_call(
        _fin_body,
        grid=(N // _BR,),
        in_specs=[
            pl.BlockSpec((_BR, W), lambda i: (i, 0)),
            pl.BlockSpec((_BR, W), lambda i: (i, 0)),
            pl.BlockSpec((1, D), lambda i: (0, 0)),
        ],
        out_specs=pl.BlockSpec((_BR, D), lambda i: (i, 0)),
        out_shape=jax.ShapeDtypeStruct((N, D), jnp.float32),
    )(parts[0], parts[1], b.reshape(1, D))


def kernel(x, edge_index, Wl1, Wr1, att1, b1, Wl2, Wr2, att2, b2):
    loop = jnp.arange(N, dtype=edge_index.dtype)
    pad = jnp.zeros((TOT - E_TOT,), dtype=edge_index.dtype)
    srcp = jnp.concatenate([edge_index[0], loop, pad]).reshape(TOT // C, C)
    dstp = jnp.concatenate([edge_index[1], loop, pad]).reshape(TOT // C, C)
    idx_pack = jnp.stack([srcp, dstp], axis=1)

    xl1, xr1 = _mm2(x, Wl1, Wr1)
    parts1 = _sc_edge(xl1, xr1, att1, idx_pack)
    xl2, xr2 = _mid(parts1, b1, Wl2, Wr2)
    parts2 = _sc_edge(xl2, xr2, att2, idx_pack)
    return _fin(parts2, b2)


# speedup vs baseline: 15.8476x; 1.1983x over previous
import functools

import jax
import jax.numpy as jnp
from jax import lax
from jax.experimental import pallas as pl
from jax.experimental.pallas import tpu as pltpu
from jax.experimental.pallas import tpu_sc as plsc

N = 10000
D = 128
E = 320000
E_TOT = E + N
NC, NS, L = 2, 16, 16
NW = NC * NS
C = 48
NCHUNK = 216
T_PER = NCHUNK * C
TOT = NW * T_PER
W = 144
ROWS_PER_TILE = N // NS
_COPY_PATTERN = tuple((i * C, C) for i in range(ROWS_PER_TILE // C)) + (
    ((ROWS_PER_TILE // C) * C, ROWS_PER_TILE % C),)


_MESH = plsc.VectorSubcoreMesh(core_axis_name="c", subcore_axis_name="s")


@functools.partial(
    pl.kernel,
    out_type=jax.ShapeDtypeStruct((NC, N, W), jnp.float32),
    mesh=_MESH,
    compiler_params=pltpu.CompilerParams(use_tc_tiling_on_sc=False,
                                         needs_layout_passes=False),
    scratch_types=[
        pltpu.VMEM((2, C), jnp.int32),
        pltpu.VMEM((2, C), jnp.int32),
        pltpu.VMEM((2, C), jnp.int32),
        pltpu.VMEM((2, C), jnp.int32),
        pltpu.VMEM((C, D), jnp.float32),
        pltpu.VMEM((C, D), jnp.float32),
        pltpu.VMEM((C, D), jnp.float32),
        pltpu.VMEM((C, D), jnp.float32),
        pltpu.VMEM((C, W), jnp.float32),
        pltpu.VMEM((C, W), jnp.float32),
        pltpu.VMEM((D,), jnp.float32),
        pltpu.VMEM_SHARED((N, W), jnp.float32),
        pltpu.SemaphoreType.DMA,
        pltpu.SemaphoreType.DMA,
        pltpu.SemaphoreType.DMA,
        pltpu.SemaphoreType.DMA,
        pltpu.SemaphoreType.DMA,
        pltpu.SemaphoreType.DMA,
        pltpu.SemaphoreType.DMA,
        pltpu.SemaphoreType.DMA,
        pltpu.SemaphoreType.DMA,
        pltpu.SemaphoreType.DMA,
    ],
)
def _sc_edge(xl_hbm, xr_hbm, att_hbm, idx_hbm, out_hbm,
             ix0, ix1, ix2, ix3, rl0, rl1, rr0, rr1, st0, st1, att_v, acc,
             si0, si1, si2, si3, sl0, sl1, sr0, sr1, ss0, ss1):
    cid = lax.axis_index("c")
    sid = lax.axis_index("s")
    wid = sid * NC + cid
    idxb = (ix0, ix1, ix2, ix3)
    sem_i = (si0, si1, si2, si3)
    rows_l = (rl0, rl1)
    rows_r = (rr0, rr1)
    sem_l = (sl0, sl1)
    sem_r = (sr0, sr1)
    staged = (st0, st1)
    sem_s = (ss0, ss1)
    base_t = wid * T_PER
    base_g = wid * NCHUNK

    def _wrap(g):
        return jnp.where(g >= NCHUNK, g - NCHUNK, g)

    def _idx_issue(g, k):
        pltpu.async_copy(idx_hbm.at[base_g + _wrap(g)], idxb[k], sem_i[k])

    def _idx_wait(g, k):
        pltpu.make_async_copy(idx_hbm.at[base_g + _wrap(g)], idxb[k],
                              sem_i[k]).wait()

    def _gather_issue(k, b):
        pltpu.async_copy(xl_hbm.at[idxb[k].at[0]], rows_l[b], sem_l[b])
        pltpu.async_copy(xr_hbm.at[idxb[k].at[1]], rows_r[b], sem_r[b])

    def _gather_wait(k, b):
        pltpu.make_async_copy(xl_hbm.at[idxb[k].at[0]], rows_l[b],
                              sem_l[b]).wait()
        pltpu.make_async_copy(xr_hbm.at[idxb[k].at[1]], rows_r[b],
                              sem_r[b]).wait()

    def _scatter_issue(k, b):
        pltpu.async_copy(staged[b], acc.at[idxb[k].at[1]], sem_s[b], add=True)

    def _scatter_wait(k, b):
        pltpu.make_async_copy(staged[b], acc.at[idxb[k].at[1]],
                              sem_s[b]).wait()

    pltpu.sync_copy(att_hbm, att_v)
    _idx_issue(0, 0)
    _idx_wait(0, 0)
    _gather_issue(0, 0)
    _idx_issue(1, 1)

    def _zrow(i, carry):
        for j in range(W // L):
            st0[i, pl.ds(j * L, L)] = jnp.zeros((L,), jnp.float32)
            st1[i, pl.ds(j * L, L)] = jnp.zeros((L,), jnp.float32)
        return carry
    lax.fori_loop(0, C, _zrow, 0)
    r0 = sid * ROWS_PER_TILE
    for off, ln in _COPY_PATTERN:
        pltpu.sync_copy(st0.at[pl.ds(0, ln)], acc.at[pl.ds(r0 + off, ln)])
    plsc.subcore_barrier()
    _scatter_issue(0, 0)
    _scatter_issue(0, 1)

    att_t0 = tuple(att_v[pl.ds(j * L, L)] for j in range(D // L))

    def quad_body(i, att_t):
        for u in range(4):
            g = 4 * i + u
            b = u % 2
            kcur = u
            knext = (u + 1) % 4
            _idx_wait(g + 1, knext)
            _gather_issue(knext, (b + 1) % 2)
            _scatter_wait(kcur, b)
            _gather_wait(kcur, b)
            ebase = base_t + g * C

            def edge_body(eh, att_tt):
                regs = []
                for s in range(4):
                    e = 4 * eh + s
                    acc0 = jnp.zeros((L,), jnp.float32)
                    acc1 = jnp.zeros((L,), jnp.float32)
                    a_regs = []
                    for j in range(D // L):
                        a = rows_l[b][e, pl.ds(j * L, L)]
                        bb = rows_r[b][e, pl.ds(j * L, L)]
                        v = a + bb
                        t = jnp.maximum(v, 0.2 * v)
                        if j % 2 == 0:
                            acc0 = acc0 + t * att_tt[j]
                        else:
                            acc1 = acc1 + t * att_tt[j]
                        a_regs.append(a)
                    logit = jnp.where(ebase + e < E_TOT,
                                      jnp.sum(acc0 + acc1), -1e30)
                    pv = jnp.exp(jnp.full((L,), logit, jnp.float32))
                    regs.append((e, a_regs, pv))
                for e, a_regs, pv in regs:
                    for j in range(D // L):
                        staged[b][e, pl.ds(j * L, L)] = a_regs[j] * pv
                    staged[b][e, pl.ds(D, L)] = pv
                return att_tt
            att_t = lax.fori_loop(0, C // 4, edge_body, att_t)
            _scatter_issue(kcur, b)
            _idx_issue(g + 2, (u + 2) % 4)
        return att_t

    lax.fori_loop(0, NCHUNK // 4, quad_body, att_t0)
    _idx_wait(1, 1)
    _gather_wait(0, 0)
    _scatter_wait(2, 0)
    _scatter_wait(3, 1)

    plsc.subcore_barrier()
    for off, ln in _COPY_PATTERN:
        pltpu.sync_copy(acc.at[pl.ds(r0 + off, ln)],
                        out_hbm.at[cid, pl.ds(r0 + off, ln)])


_BR = 1000


def _mm2_body(x_ref, wl_ref, wr_ref, xl_ref, xr_ref):
    xb = x_ref[...]
    xl_ref[...] = jnp.dot(xb, wl_ref[...], preferred_element_type=jnp.float32)
    xr_ref[...] = jnp.dot(xb, wr_ref[...], preferred_element_type=jnp.float32)


def _mm2(x, wl, wr):
    return pl.pallas_call(
        _mm2_body,
        grid=(N // _BR,),
        in_specs=[
            pl.BlockSpec((_BR, D), lambda i: (i, 0)),
            pl.BlockSpec((D, D), lambda i: (0, 0)),
            pl.BlockSpec((D, D), lambda i: (0, 0)),
        ],
        out_specs=[
            pl.BlockSpec((_BR, D), lambda i: (i, 0)),
            pl.BlockSpec((_BR, D), lambda i: (i, 0)),
        ],
        out_shape=[
            jax.ShapeDtypeStruct((N, D), jnp.float32),
            jax.ShapeDtypeStruct((N, D), jnp.float32),
        ],
    )(x, wl, wr)


def _combine(p0, p1):
    num = p0[:, :D] + p1[:, :D]
    den = p0[:, D:D + 1] + p1[:, D:D + 1]
    return num / (den + 1e-16)


def _mid_body(p0_ref, p1_ref, b_ref, wl_ref, wr_ref, xl_ref, xr_ref):
    h = jnp.maximum(_combine(p0_ref[...], p1_ref[...]) + b_ref[...], 0.0)
    xl_ref[...] = jnp.dot(h, wl_ref[...], preferred_element_type=jnp.float32)
    xr_ref[...] = jnp.dot(h, wr_ref[...], preferred_element_type=jnp.float32)


def _mid(parts, b, wl, wr):
    return pl.pallas_call(
        _mid_body,
        grid=(N // _BR,),
        in_specs=[
            pl.BlockSpec((_BR, W), lambda i: (i, 0)),
            pl.BlockSpec((_BR, W), lambda i: (i, 0)),
            pl.BlockSpec((1, D), lambda i: (0, 0)),
            pl.BlockSpec((D, D), lambda i: (0, 0)),
            pl.BlockSpec((D, D), lambda i: (0, 0)),
        ],
        out_specs=[
            pl.BlockSpec((_BR, D), lambda i: (i, 0)),
            pl.BlockSpec((_BR, D), lambda i: (i, 0)),
        ],
        out_shape=[
            jax.ShapeDtypeStruct((N, D), jnp.float32),
            jax.ShapeDtypeStruct((N, D), jnp.float32),
        ],
    )(parts[0], parts[1], b.reshape(1, D), wl, wr)


def _fin_body(p0_ref, p1_ref, b_ref, o_ref):
    o_ref[...] = _combine(p0_ref[...], p1_ref[...]) + b_ref[...]


def _fin(parts, b):
    return pl.pallas_call(
        _fin_body,
        grid=(N // _BR,),
        in_specs=[
            pl.BlockSpec((_BR, W), lambda i: (i, 0)),
            pl.BlockSpec((_BR, W), lambda i: (i, 0)),
            pl.BlockSpec((1, D), lambda i: (0, 0)),
        ],
        out_specs=pl.BlockSpec((_BR, D), lambda i: (i, 0)),
        out_shape=jax.ShapeDtypeStruct((N, D), jnp.float32),
    )(parts[0], parts[1], b.reshape(1, D))


def kernel(x, edge_index, Wl1, Wr1, att1, b1, Wl2, Wr2, att2, b2):
    loop = jnp.arange(N, dtype=edge_index.dtype)
    pad = jnp.zeros((TOT - E_TOT,), dtype=edge_index.dtype)
    srcp = jnp.concatenate([edge_index[0], loop, pad]).reshape(TOT // C, C)
    dstp = jnp.concatenate([edge_index[1], loop, pad]).reshape(TOT // C, C)
    idx_pack = jnp.stack([srcp, dstp], axis=1)

    xl1, xr1 = _mm2(x, Wl1, Wr1)
    parts1 = _sc_edge(xl1, xr1, att1, idx_pack)
    xl2, xr2 = _mid(parts1, b1, Wl2, Wr2)
    parts2 = _sc_edge(xl2, xr2, att2, idx_pack)
    return _fin(parts2, b2)


# speedup vs baseline: 16.0044x; 1.0099x over previous
import functools

import jax
import jax.numpy as jnp
from jax import lax
from jax.experimental import pallas as pl
from jax.experimental.pallas import tpu as pltpu
from jax.experimental.pallas import tpu_sc as plsc

N = 10000
D = 128
E = 320000
E_TOT = E + N
NC, NS, L = 2, 16, 16
NW = NC * NS
C = 48
NCHUNK = 216
T_PER = NCHUNK * C
TOT = NW * T_PER
W = 144
ROWS_PER_TILE = N // NS
_COPY_PATTERN = tuple((i * C, C) for i in range(ROWS_PER_TILE // C)) + (
    ((ROWS_PER_TILE // C) * C, ROWS_PER_TILE % C),)


_MESH = plsc.VectorSubcoreMesh(core_axis_name="c", subcore_axis_name="s")


@functools.partial(
    pl.kernel,
    out_type=jax.ShapeDtypeStruct((NC, N, W), jnp.float32),
    mesh=_MESH,
    compiler_params=pltpu.CompilerParams(use_tc_tiling_on_sc=False,
                                         needs_layout_passes=False),
    scratch_types=[
        pltpu.VMEM((2, C), jnp.int32),
        pltpu.VMEM((2, C), jnp.int32),
        pltpu.VMEM((2, C), jnp.int32),
        pltpu.VMEM((2, C), jnp.int32),
        pltpu.VMEM((C, D), jnp.float32),
        pltpu.VMEM((C, D), jnp.float32),
        pltpu.VMEM((C, D), jnp.float32),
        pltpu.VMEM((C, D), jnp.float32),
        pltpu.VMEM((C, W), jnp.float32),
        pltpu.VMEM((C, W), jnp.float32),
        pltpu.VMEM((D,), jnp.float32),
        pltpu.VMEM_SHARED((N, W), jnp.float32),
        pltpu.SemaphoreType.DMA,
        pltpu.SemaphoreType.DMA,
        pltpu.SemaphoreType.DMA,
        pltpu.SemaphoreType.DMA,
        pltpu.SemaphoreType.DMA,
        pltpu.SemaphoreType.DMA,
        pltpu.SemaphoreType.DMA,
        pltpu.SemaphoreType.DMA,
        pltpu.SemaphoreType.DMA,
        pltpu.SemaphoreType.DMA,
    ],
)
def _sc_edge(xl_hbm, xr_hbm, att_hbm, idx_hbm, out_hbm,
             ix0, ix1, ix2, ix3, rl0, rl1, rr0, rr1, st0, st1, att_v, acc,
             si0, si1, si2, si3, sl0, sl1, sr0, sr1, ss0, ss1):
    cid = lax.axis_index("c")
    sid = lax.axis_index("s")
    wid = sid * NC + cid
    idxb = (ix0, ix1, ix2, ix3)
    sem_i = (si0, si1, si2, si3)
    rows_l = (rl0, rl1)
    rows_r = (rr0, rr1)
    sem_l = (sl0, sl1)
    sem_r = (sr0, sr1)
    staged = (st0, st1)
    sem_s = (ss0, ss1)
    base_t = wid * T_PER
    base_g = wid * NCHUNK

    def _wrap(g):
        return jnp.where(g >= NCHUNK, g - NCHUNK, g)

    def _idx_issue(g, k):
        pltpu.async_copy(idx_hbm.at[base_g + _wrap(g)], idxb[k], sem_i[k])

    def _idx_wait(g, k):
        pltpu.make_async_copy(idx_hbm.at[base_g + _wrap(g)], idxb[k],
                              sem_i[k]).wait()

    def _gather_issue(k, b):
        pltpu.async_copy(xl_hbm.at[idxb[k].at[0]], rows_l[b], sem_l[b])
        pltpu.async_copy(xr_hbm.at[idxb[k].at[1]], rows_r[b], sem_r[b])

    def _gather_wait(k, b):
        pltpu.make_async_copy(xl_hbm.at[idxb[k].at[0]], rows_l[b],
                              sem_l[b]).wait()
        pltpu.make_async_copy(xr_hbm.at[idxb[k].at[1]], rows_r[b],
                              sem_r[b]).wait()

    def _scatter_issue(k, b):
        pltpu.async_copy(staged[b], acc.at[idxb[k].at[1]], sem_s[b], add=True)

    def _scatter_wait(k, b):
        pltpu.make_async_copy(staged[b], acc.at[idxb[k].at[1]],
                              sem_s[b]).wait()

    pltpu.sync_copy(att_hbm, att_v)
    _idx_issue(0, 0)
    _idx_wait(0, 0)
    _gather_issue(0, 0)
    _idx_issue(1, 1)

    def _zrow(i, carry):
        for j in range(W // L):
            st0[i, pl.ds(j * L, L)] = jnp.zeros((L,), jnp.float32)
            st1[i, pl.ds(j * L, L)] = jnp.zeros((L,), jnp.float32)
        return carry
    lax.fori_loop(0, C, _zrow, 0)
    r0 = sid * ROWS_PER_TILE
    for off, ln in _COPY_PATTERN:
        pltpu.sync_copy(st0.at[pl.ds(0, ln)], acc.at[pl.ds(r0 + off, ln)])
    plsc.subcore_barrier()
    _scatter_issue(0, 0)
    _scatter_issue(0, 1)

    att_t0 = tuple(att_v[pl.ds(j * L, L)] for j in range(D // L))

    def quad_body(i, att_t):
        for u in range(4):
            g = 4 * i + u
            b = u % 2
            kcur = u
            knext = (u + 1) % 4
            _idx_wait(g + 1, knext)
            _gather_issue(knext, (b + 1) % 2)
            _scatter_wait(kcur, b)
            _gather_wait(kcur, b)
            ebase = base_t + g * C

            def edge_body(eh, att_tt):
                regs = []
                for s in range(8):
                    e = 8 * eh + s
                    acc0 = jnp.zeros((L,), jnp.float32)
                    acc1 = jnp.zeros((L,), jnp.float32)
                    a_regs = []
                    for j in range(D // L):
                        a = rows_l[b][e, pl.ds(j * L, L)]
                        bb = rows_r[b][e, pl.ds(j * L, L)]
                        v = a + bb
                        t = jnp.maximum(v, 0.2 * v)
                        if j % 2 == 0:
                            acc0 = acc0 + t * att_tt[j]
                        else:
                            acc1 = acc1 + t * att_tt[j]
                        a_regs.append(a)
                    logit = jnp.where(ebase + e < E_TOT,
                                      jnp.sum(acc0 + acc1), -1e30)
                    pv = jnp.exp(jnp.full((L,), logit, jnp.float32))
                    regs.append((e, a_regs, pv))
                for e, a_regs, pv in regs:
                    for j in range(D // L):
                        staged[b][e, pl.ds(j * L, L)] = a_regs[j] * pv
                    staged[b][e, pl.ds(D, L)] = pv
                return att_tt
            att_t = lax.fori_loop(0, C // 8, edge_body, att_t)
            _scatter_issue(kcur, b)
            _idx_issue(g + 2, (u + 2) % 4)
        return att_t

    lax.fori_loop(0, NCHUNK // 4, quad_body, att_t0)
    _idx_wait(1, 1)
    _gather_wait(0, 0)
    _scatter_wait(2, 0)
    _scatter_wait(3, 1)

    plsc.subcore_barrier()
    for off, ln in _COPY_PATTERN:
        pltpu.sync_copy(acc.at[pl.ds(r0 + off, ln)],
                        out_hbm.at[cid, pl.ds(r0 + off, ln)])


_BR = 1000


def _mm2_body(x_ref, wl_ref, wr_ref, xl_ref, xr_ref):
    xb = x_ref[...]
    xl_ref[...] = jnp.dot(xb, wl_ref[...], preferred_element_type=jnp.float32)
    xr_ref[...] = jnp.dot(xb, wr_ref[...], preferred_element_type=jnp.float32)


def _mm2(x, wl, wr):
    return pl.pallas_call(
        _mm2_body,
        grid=(N // _BR,),
        in_specs=[
            pl.BlockSpec((_BR, D), lambda i: (i, 0)),
            pl.BlockSpec((D, D), lambda i: (0, 0)),
            pl.BlockSpec((D, D), lambda i: (0, 0)),
        ],
        out_specs=[
            pl.BlockSpec((_BR, D), lambda i: (i, 0)),
            pl.BlockSpec((_BR, D), lambda i: (i, 0)),
        ],
        out_shape=[
            jax.ShapeDtypeStruct((N, D), jnp.float32),
            jax.ShapeDtypeStruct((N, D), jnp.float32),
        ],
    )(x, wl, wr)


def _combine(p0, p1):
    num = p0[:, :D] + p1[:, :D]
    den = p0[:, D:D + 1] + p1[:, D:D + 1]
    return num / (den + 1e-16)


def _mid_body(p0_ref, p1_ref, b_ref, wl_ref, wr_ref, xl_ref, xr_ref):
    h = jnp.maximum(_combine(p0_ref[...], p1_ref[...]) + b_ref[...], 0.0)
    xl_ref[...] = jnp.dot(h, wl_ref[...], preferred_element_type=jnp.float32)
    xr_ref[...] = jnp.dot(h, wr_ref[...], preferred_element_type=jnp.float32)


def _mid(parts, b, wl, wr):
    return pl.pallas_call(
        _mid_body,
        grid=(N // _BR,),
        in_specs=[
            pl.BlockSpec((_BR, W), lambda i: (i, 0)),
            pl.BlockSpec((_BR, W), lambda i: (i, 0)),
            pl.BlockSpec((1, D), lambda i: (0, 0)),
            pl.BlockSpec((D, D), lambda i: (0, 0)),
            pl.BlockSpec((D, D), lambda i: (0, 0)),
        ],
        out_specs=[
            pl.BlockSpec((_BR, D), lambda i: (i, 0)),
            pl.BlockSpec((_BR, D), lambda i: (i, 0)),
        ],
        out_shape=[
            jax.ShapeDtypeStruct((N, D), jnp.float32),
            jax.ShapeDtypeStruct((N, D), jnp.float32),
        ],
    )(parts[0], parts[1], b.reshape(1, D), wl, wr)


def _fin_body(p0_ref, p1_ref, b_ref, o_ref):
    o_ref[...] = _combine(p0_ref[...], p1_ref[...]) + b_ref[...]


def _fin(parts, b):
    return pl.pallas_call(
        _fin_body,
        grid=(N // _BR,),
        in_specs=[
            pl.BlockSpec((_BR, W), lambda i: (i, 0)),
            pl.BlockSpec((_BR, W), lambda i: (i, 0)),
            pl.BlockSpec((1, D), lambda i: (0, 0)),
        ],
        out_specs=pl.BlockSpec((_BR, D), lambda i: (i, 0)),
        out_shape=jax.ShapeDtypeStruct((N, D), jnp.float32),
    )(parts[0], parts[1], b.reshape(1, D))


def kernel(x, edge_index, Wl1, Wr1, att1, b1, Wl2, Wr2, att2, b2):
    loop = jnp.arange(N, dtype=edge_index.dtype)
    pad = jnp.zeros((TOT - E_TOT,), dtype=edge_index.dtype)
    srcp = jnp.concatenate([edge_index[0], loop, pad]).reshape(TOT // C, C)
    dstp = jnp.concatenate([edge_index[1], loop, pad]).reshape(TOT // C, C)
    idx_pack = jnp.stack([srcp, dstp], axis=1)

    xl1, xr1 = _mm2(x, Wl1, Wr1)
    parts1 = _sc_edge(xl1, xr1, att1, idx_pack)
    xl2, xr2 = _mid(parts1, b1, Wl2, Wr2)
    parts2 = _sc_edge(xl2, xr2, att2, idx_pack)
    return _fin(parts2, b2)
